# baseline (speedup 1.0000x reference)
import functools

import jax
import jax.numpy as jnp
from jax import lax
from jax.experimental import pallas as pl
from jax.experimental.pallas import tpu as pltpu
from jax.experimental.pallas import tpu_sc as plsc

N = 10000
D = 128
H = D // 2
E = 320000
NEG_SLOPE = 0.01

NC = 2
NS = 16
NW = NC * NS

WIN = 128

NWIN_D = -(-E // (NW * WIN))
EP_D = NW * NWIN_D * WIN
DR_D = 240
NP_D = N + DR_D
ZR_D = NP_D // NS

NWIN = -(-E // (NS * WIN))
EP = NS * NWIN * WIN
DR = 112
NP = N + DR
ZR = NP // NS

_mesh = plsc.VectorSubcoreMesh(core_axis_name="c", subcore_axis_name="s")


@functools.partial(
    pl.kernel,
    out_type=jax.ShapeDtypeStruct((NC, NP_D), jnp.float32),
    mesh=_mesh,
    scratch_types=[
        pltpu.VMEM((NWIN_D, WIN), jnp.int32),
        pltpu.VMEM((WIN,), jnp.float32),
        pltpu.VMEM_SHARED((NP_D,), jnp.float32),
    ],
)
def _deg_kernel(dst_hbm, zeros_hbm, out_hbm, idx_v, ones_v, acc):
    cid = lax.axis_index("c")
    sid = lax.axis_index("s")
    wid = sid * NC + cid
    pltpu.sync_copy(zeros_hbm, acc.at[pl.ds(sid * ZR_D, ZR_D)])
    for i in range(WIN // 16):
        ones_v[pl.ds(i * 16, 16)] = jnp.ones((16,), jnp.float32)
    pltpu.sync_copy(dst_hbm.at[wid], idx_v)
    plsc.subcore_barrier()

    def body(j, carry):
        pltpu.sync_copy(ones_v, acc.at[idx_v.at[j]], add=True)
        return carry

    lax.fori_loop(0, NWIN_D, body, 0)
    plsc.subcore_barrier()
    pltpu.sync_copy(acc.at[pl.ds(sid * ZR_D, ZR_D)], out_hbm.at[cid, pl.ds(sid * ZR_D, ZR_D)])


NB = 6
GAH = 2
SDR = 4


@functools.partial(
    pl.kernel,
    out_type=jax.ShapeDtypeStruct((NC, NP, H), jnp.float32),
    mesh=_mesh,
    scratch_types=[
        pltpu.VMEM((NWIN, WIN), jnp.int32),
        pltpu.VMEM((NWIN, WIN), jnp.int32),
        pltpu.VMEM((NB, WIN, H), jnp.float32),
        pltpu.VMEM_SHARED((NP, H), jnp.float32),
        pltpu.SemaphoreType.DMA((NB,)),
        pltpu.SemaphoreType.DMA((NB,)),
    ],
    compiler_params=pltpu.CompilerParams(use_tc_tiling_on_sc=False),
)
def _agg_kernel(y_hbm, src_hbm, dst_hbm, zeros_hbm, out_hbm, src_v, dst_v,
                rows_v, acc, gsem, ssem):
    cid = lax.axis_index("c")
    sid = lax.axis_index("s")

    def gather(j):
        b = j % NB
        return pltpu.make_async_copy(y_hbm.at[src_v.at[j]], rows_v.at[b],
                                     gsem.at[b])

    def scatter(j):
        b = j % NB
        return pltpu.make_async_copy(rows_v.at[b], acc.at[dst_v.at[j]],
                                     ssem.at[b])

    pltpu.sync_copy(zeros_hbm, acc.at[pl.ds(sid * ZR, ZR)])
    pltpu.sync_copy(src_hbm.at[cid, sid], src_v)
    pltpu.sync_copy(dst_hbm.at[sid], dst_v)
    plsc.subcore_barrier()

    for j in range(GAH):
        gather(j).start()

    def body(j, carry):
        @pl.when(j >= SDR)
        def _():
            scatter(j - SDR).wait()

        gather(j).wait()
        scatter(j).start(add=True)

        @pl.when(j + GAH < NWIN)
        def _():
            gather(j + GAH).start()

        return carry

    lax.fori_loop(0, NWIN, body, 0)
    for j in range(NWIN - SDR, NWIN):
        scatter(j).wait()
    plsc.subcore_barrier()
    pltpu.sync_copy(acc.at[pl.ds(sid * ZR, ZR)], out_hbm.at[cid, pl.ds(sid * ZR, ZR)])


def _lrelu(t):
    return jnp.where(t >= 0, t, NEG_SLOPE * t)


RB = 1000


def _tca_body(p_ref, x_ref, w1_ref, y1_ref, dis_ref):
    dis = lax.rsqrt(p_ref[...] + 1.0)
    xw = jnp.dot(x_ref[...], w1_ref[...], preferred_element_type=jnp.float32)
    y1_ref[...] = dis * xw
    dis_ref[...] = dis


def _tcb_body(agg_ref, y1_ref, dis_ref, b1_ref, w2_ref, h1_ref, y2_ref):
    dis = dis_ref[...]
    agg = jnp.concatenate([agg_ref[0], agg_ref[1]], axis=1)
    out1 = dis * (agg + y1_ref[...]) + b1_ref[...]
    h1 = _lrelu(out1)
    h1_ref[...] = h1
    y2_ref[...] = dis * jnp.dot(h1, w2_ref[...], preferred_element_type=jnp.float32)


def _tcc_body(agg_ref, y2_ref, h1_ref, dis_ref, b2_ref, wm1_ref, bm1_ref,
              wm2_ref, bm2_ref, out_ref):
    dis = dis_ref[...]
    agg = jnp.concatenate([agg_ref[0], agg_ref[1]], axis=1)
    g = dis * (agg + y2_ref[...]) + b2_ref[...]
    t = _lrelu(g + h1_ref[...])
    u = _lrelu(jnp.dot(t, wm1_ref[...], preferred_element_type=jnp.float32) + bm1_ref[...])
    mlp = jnp.dot(u, wm2_ref[...], preferred_element_type=jnp.float32) + bm2_ref[...]
    out_ref[...] = _lrelu(mlp + t)


def _row_spec(w):
    return pl.BlockSpec((RB, w), lambda i: (i, 0))


def _full_spec(shape):
    return pl.BlockSpec(shape, lambda i: (0,) * len(shape))


_GRID = N // RB

_tca = pl.pallas_call(
    _tca_body,
    grid=(_GRID,),
    in_specs=[_row_spec(1), _row_spec(D), _full_spec((D, D))],
    out_specs=[_row_spec(D), _row_spec(1)],
    out_shape=[
        jax.ShapeDtypeStruct((N, D), jnp.float32),
        jax.ShapeDtypeStruct((N, 1), jnp.float32),
    ],
)

_tcb = pl.pallas_call(
    _tcb_body,
    grid=(_GRID,),
    in_specs=[
        pl.BlockSpec((NC, RB, H), lambda i: (0, i, 0)),
        _row_spec(D), _row_spec(1), _full_spec((1, D)), _full_spec((D, D)),
    ],
    out_specs=[_row_spec(D), _row_spec(D)],
    out_shape=[
        jax.ShapeDtypeStruct((N, D), jnp.float32),
        jax.ShapeDtypeStruct((N, D), jnp.float32),
    ],
)

_tcc = pl.pallas_call(
    _tcc_body,
    grid=(_GRID,),
    in_specs=[
        pl.BlockSpec((NC, RB, H), lambda i: (0, i, 0)),
        _row_spec(D), _row_spec(D), _row_spec(1),
        _full_spec((1, D)), _full_spec((D, D)), _full_spec((1, D)),
        _full_spec((D, D)), _full_spec((1, D)),
    ],
    out_specs=_row_spec(D),
    out_shape=jax.ShapeDtypeStruct((N, D), jnp.float32),
)


def kernel(x, edge_index, W1, b1, W2, b2, Wm1, bm1, Wm2, bm2):
    ei = edge_index.astype(jnp.int32)

    pad_d = jnp.arange(EP_D - E, dtype=jnp.int32)
    dst_d = jnp.concatenate([ei[1], N + pad_d % DR_D]).reshape(NW, NWIN_D, WIN)

    pad_a = jnp.arange(EP - E, dtype=jnp.int32)
    src_p = jnp.concatenate([ei[0], pad_a % N])
    dst_p = jnp.concatenate([ei[1], N + pad_a % DR]).reshape(NS, NWIN, WIN)
    src2 = 2 * src_p
    src_c = jnp.stack([src2, src2 + 1]).reshape(NC, NS, NWIN, WIN)

    zeros1 = jnp.zeros((ZR_D,), jnp.float32)
    zeros2 = jnp.zeros((ZR, H), jnp.float32)

    deg_parts = _deg_kernel(dst_d, zeros1)
    p = (deg_parts[0, :N] + deg_parts[1, :N])[:, None]

    y1, dis = _tca(p, x, W1)
    agg1 = _agg_kernel(y1.reshape(2 * N, H), src_c, dst_p, zeros2)[:, :N]
    h1, y2 = _tcb(agg1, y1, dis, b1.reshape(1, D), W2)
    agg2 = _agg_kernel(y2.reshape(2 * N, H), src_c, dst_p, zeros2)[:, :N]
    out = _tcc(agg2, y2, h1, dis, b2.reshape(1, D), Wm1, bm1.reshape(1, D),
               Wm2, bm2.reshape(1, D))
    return out


# speedup vs baseline: 1.1594x; 1.1594x over previous
import functools

import jax
import jax.numpy as jnp
from jax import lax
from jax.experimental import pallas as pl
from jax.experimental.pallas import tpu as pltpu
from jax.experimental.pallas import tpu_sc as plsc

N = 10000
D = 128
H = D // 2
E = 320000
NEG_SLOPE = 0.01

NC = 2
NS = 16
NW = NC * NS

WIN = 128

NWIN_D = -(-E // (NW * WIN))
EP_D = NW * NWIN_D * WIN
DR_D = 240
NP_D = N + DR_D
ZR_D = NP_D // NS

NWIN = -(-E // (NS * WIN))
EP = NS * NWIN * WIN
DR = 112
NP = N + DR
ZR = NP // NS

_mesh = plsc.VectorSubcoreMesh(core_axis_name="c", subcore_axis_name="s")


@functools.partial(
    pl.kernel,
    out_type=jax.ShapeDtypeStruct((NC, NP_D), jnp.float32),
    mesh=_mesh,
    scratch_types=[
        pltpu.VMEM((NWIN_D, WIN), jnp.int32),
        pltpu.VMEM((WIN,), jnp.float32),
        pltpu.VMEM_SHARED((NP_D,), jnp.float32),
    ],
)
def _deg_kernel(dst_hbm, zeros_hbm, out_hbm, idx_v, ones_v, acc):
    cid = lax.axis_index("c")
    sid = lax.axis_index("s")
    wid = sid * NC + cid
    pltpu.sync_copy(zeros_hbm, acc.at[pl.ds(sid * ZR_D, ZR_D)])
    for i in range(WIN // 16):
        ones_v[pl.ds(i * 16, 16)] = jnp.ones((16,), jnp.float32)
    pltpu.sync_copy(dst_hbm.at[wid], idx_v)
    plsc.subcore_barrier()

    def body(j, carry):
        pltpu.sync_copy(ones_v, acc.at[idx_v.at[j]], add=True)
        return carry

    lax.fori_loop(0, NWIN_D, body, 0)
    plsc.subcore_barrier()
    pltpu.sync_copy(acc.at[pl.ds(sid * ZR_D, ZR_D)], out_hbm.at[cid, pl.ds(sid * ZR_D, ZR_D)])


NB = 6
GAH = 4
SDR = 2


@functools.partial(
    pl.kernel,
    out_type=jax.ShapeDtypeStruct((NC, NP, H), jnp.float32),
    mesh=_mesh,
    scratch_types=[
        pltpu.VMEM((NWIN, WIN), jnp.int32),
        pltpu.VMEM((NWIN, WIN), jnp.int32),
        pltpu.VMEM((NB, WIN, H), jnp.float32),
        pltpu.VMEM_SHARED((NP, H), jnp.float32),
        pltpu.SemaphoreType.DMA((NB,)),
        pltpu.SemaphoreType.DMA((NB,)),
    ],
    compiler_params=pltpu.CompilerParams(use_tc_tiling_on_sc=False),
)
def _agg_kernel(y_hbm, src_hbm, dst_hbm, zeros_hbm, out_hbm, src_v, dst_v,
                rows_v, acc, gsem, ssem):
    cid = lax.axis_index("c")
    sid = lax.axis_index("s")

    def gather(j):
        b = j % NB
        return pltpu.make_async_copy(y_hbm.at[src_v.at[j]], rows_v.at[b],
                                     gsem.at[b])

    def scatter(j):
        b = j % NB
        return pltpu.make_async_copy(rows_v.at[b], acc.at[dst_v.at[j]],
                                     ssem.at[b])

    pltpu.sync_copy(zeros_hbm, acc.at[pl.ds(sid * ZR, ZR)])
    pltpu.sync_copy(src_hbm.at[cid, sid], src_v)
    pltpu.sync_copy(dst_hbm.at[sid], dst_v)
    plsc.subcore_barrier()

    for j in range(GAH):
        gather(j).start()

    def body(j, carry):
        @pl.when(j >= SDR)
        def _():
            scatter(j - SDR).wait()

        gather(j).wait()
        scatter(j).start(add=True)

        @pl.when(j + GAH < NWIN)
        def _():
            gather(j + GAH).start()

        return carry

    lax.fori_loop(0, NWIN, body, 0)
    for j in range(NWIN - SDR, NWIN):
        scatter(j).wait()
    plsc.subcore_barrier()
    pltpu.sync_copy(acc.at[pl.ds(sid * ZR, ZR)], out_hbm.at[cid, pl.ds(sid * ZR, ZR)])


def _lrelu(t):
    return jnp.where(t >= 0, t, NEG_SLOPE * t)


RB = 1000


def _tca_body(p_ref, x_ref, w1_ref, y1_ref, dis_ref):
    dis = lax.rsqrt(p_ref[...] + 1.0)
    xw = jnp.dot(x_ref[...], w1_ref[...], preferred_element_type=jnp.float32)
    y1_ref[...] = dis * xw
    dis_ref[...] = dis


def _tcb_body(agg_ref, y1_ref, dis_ref, b1_ref, w2_ref, h1_ref, y2_ref):
    dis = dis_ref[...]
    agg = jnp.concatenate([agg_ref[0], agg_ref[1]], axis=1)
    out1 = dis * (agg + y1_ref[...]) + b1_ref[...]
    h1 = _lrelu(out1)
    h1_ref[...] = h1
    y2_ref[...] = dis * jnp.dot(h1, w2_ref[...], preferred_element_type=jnp.float32)


def _tcc_body(agg_ref, y2_ref, h1_ref, dis_ref, b2_ref, wm1_ref, bm1_ref,
              wm2_ref, bm2_ref, out_ref):
    dis = dis_ref[...]
    agg = jnp.concatenate([agg_ref[0], agg_ref[1]], axis=1)
    g = dis * (agg + y2_ref[...]) + b2_ref[...]
    t = _lrelu(g + h1_ref[...])
    u = _lrelu(jnp.dot(t, wm1_ref[...], preferred_element_type=jnp.float32) + bm1_ref[...])
    mlp = jnp.dot(u, wm2_ref[...], preferred_element_type=jnp.float32) + bm2_ref[...]
    out_ref[...] = _lrelu(mlp + t)


def _row_spec(w):
    return pl.BlockSpec((RB, w), lambda i: (i, 0))


def _full_spec(shape):
    return pl.BlockSpec(shape, lambda i: (0,) * len(shape))


_GRID = N // RB

_tca = pl.pallas_call(
    _tca_body,
    grid=(_GRID,),
    in_specs=[_row_spec(1), _row_spec(D), _full_spec((D, D))],
    out_specs=[_row_spec(D), _row_spec(1)],
    out_shape=[
        jax.ShapeDtypeStruct((N, D), jnp.float32),
        jax.ShapeDtypeStruct((N, 1), jnp.float32),
    ],
)

_tcb = pl.pallas_call(
    _tcb_body,
    grid=(_GRID,),
    in_specs=[
        pl.BlockSpec((NC, RB, H), lambda i: (0, i, 0)),
        _row_spec(D), _row_spec(1), _full_spec((1, D)), _full_spec((D, D)),
    ],
    out_specs=[_row_spec(D), _row_spec(D)],
    out_shape=[
        jax.ShapeDtypeStruct((N, D), jnp.float32),
        jax.ShapeDtypeStruct((N, D), jnp.float32),
    ],
)

_tcc = pl.pallas_call(
    _tcc_body,
    grid=(_GRID,),
    in_specs=[
        pl.BlockSpec((NC, RB, H), lambda i: (0, i, 0)),
        _row_spec(D), _row_spec(D), _row_spec(1),
        _full_spec((1, D)), _full_spec((D, D)), _full_spec((1, D)),
        _full_spec((D, D)), _full_spec((1, D)),
    ],
    out_specs=_row_spec(D),
    out_shape=jax.ShapeDtypeStruct((N, D), jnp.float32),
)


def kernel(x, edge_index, W1, b1, W2, b2, Wm1, bm1, Wm2, bm2):
    ei = edge_index.astype(jnp.int32)

    pad_d = jnp.arange(EP_D - E, dtype=jnp.int32)
    dst_d = jnp.concatenate([ei[1], N + pad_d % DR_D]).reshape(NW, NWIN_D, WIN)

    pad_a = jnp.arange(EP - E, dtype=jnp.int32)
    src_p = jnp.concatenate([ei[0], pad_a % N])
    dst_p = jnp.concatenate([ei[1], N + pad_a % DR]).reshape(NS, NWIN, WIN)
    src2 = 2 * src_p
    src_c = jnp.stack([src2, src2 + 1]).reshape(NC, NS, NWIN, WIN)

    zeros1 = jnp.zeros((ZR_D,), jnp.float32)
    zeros2 = jnp.zeros((ZR, H), jnp.float32)

    deg_parts = _deg_kernel(dst_d, zeros1)
    p = (deg_parts[0, :N] + deg_parts[1, :N])[:, None]

    y1, dis = _tca(p, x, W1)
    agg1 = _agg_kernel(y1.reshape(2 * N, H), src_c, dst_p, zeros2)[:, :N]
    h1, y2 = _tcb(agg1, y1, dis, b1.reshape(1, D), W2)
    agg2 = _agg_kernel(y2.reshape(2 * N, H), src_c, dst_p, zeros2)[:, :N]
    out = _tcc(agg2, y2, h1, dis, b2.reshape(1, D), Wm1, bm1.reshape(1, D),
               Wm2, bm2.reshape(1, D))
    return out
